# baseline (speedup 1.0000x reference)
import functools

import jax
import jax.numpy as jnp
from jax import lax
from jax.experimental import pallas as pl
from jax.experimental.pallas import tpu as pltpu
from jax.experimental.pallas import tpu_sc as plsc

NUM_NODES = 1000000
MEMORY_DIM = 64
BATCH = 16384

_NC = 2
_NS = 16
_NW = _NC * _NS
_BPW = BATCH // _NW
_CHUNK = 128
_NCHUNK = _BPW // _CHUNK

_mesh = plsc.VectorSubcoreMesh(core_axis_name="c", subcore_axis_name="s")


@functools.partial(
    pl.kernel,
    mesh=_mesh,
    compiler_params=pltpu.CompilerParams(use_tc_tiling_on_sc=False),
    out_type=[
        jax.ShapeDtypeStruct((BATCH, MEMORY_DIM), jnp.float32),
        jax.ShapeDtypeStruct((BATCH,), jnp.int32),
    ],
    scratch_types=[
        pltpu.VMEM((_NCHUNK, _CHUNK), jnp.int32),
        pltpu.VMEM((_BPW, MEMORY_DIM), jnp.float32),
        pltpu.VMEM((_BPW,), jnp.int32),
        pltpu.SemaphoreType.DMA,
        pltpu.SemaphoreType.DMA,
    ],
)
def _dual_gather(mem_hbm, lu_hbm, idx_hbm, mem_out, lu_out,
                 idx_v, rows_v, lu_v, sem_rows, sem_lu):
    wid = lax.axis_index("s") * _NC + lax.axis_index("c")
    base = wid * _BPW
    pltpu.sync_copy(idx_hbm.at[wid], idx_v)
    copies = []
    for j in range(_NCHUNK):
        copies.append(pltpu.async_copy(
            mem_hbm.at[idx_v.at[j]],
            rows_v.at[pl.ds(j * _CHUNK, _CHUNK)],
            sem_rows))
        copies.append(pltpu.async_copy(
            lu_hbm.at[idx_v.at[j]],
            lu_v.at[pl.ds(j * _CHUNK, _CHUNK)],
            sem_lu))
    for c in copies:
        c.wait()
    pltpu.sync_copy(rows_v, mem_out.at[pl.ds(base, _BPW)])
    pltpu.sync_copy(lu_v, lu_out.at[pl.ds(base, _BPW)])


def kernel(memory, last_update, n_id):
    idx = n_id.astype(jnp.int32).reshape(_NW, _NCHUNK, _CHUNK)
    lu32 = last_update.astype(jnp.int32)
    mem_out, lu_out = _dual_gather(memory, lu32, idx)
    return (mem_out, lu_out.astype(last_update.dtype))
